# Initial kernel scaffold; baseline (speedup 1.0000x reference)
import jax
import jax.numpy as jnp
from jax.experimental import pallas as pl


def kernel(u, pos, variables, enc_W1, enc_b1, enc_W2, enc_b2, msg_W1, msg_b1, msg_W2, msg_b2, upd_W1, upd_b1, upd_W2, upd_b2, dec_W, dec_b, edge_index):
    raise NotImplementedError("write your pallas kernel here")


# speedup vs baseline: 4.3100x; 4.3100x over previous
import functools

import jax
import jax.numpy as jnp
from jax import lax
from jax.experimental import pallas as pl
from jax.experimental.pallas import tpu as pltpu
from jax.experimental.pallas import tpu_sc as plsc

N = 10000
E = 320000
TW = 25
NV = 1
D = 128
L = 6

NC = 2
NS = 16
NW = NC * NS
EPW = E // NW
CH = 80
NCHUNK = EPW // CH
NP = 10240
NPS = NP // NS

BN = 1000
GN = N // BN
BE = 2000
GE = E // BE


def _silu(x):
    return x * jax.nn.sigmoid(x)


def _mm(a, b):
    return jax.lax.dot_general(a, b, (((1,), (0,)), ((), ())),
                               preferred_element_type=jnp.float32)


_MESH = plsc.VectorSubcoreMesh(core_axis_name="c", subcore_axis_name="s")


@functools.partial(
    pl.kernel,
    out_type=jax.ShapeDtypeStruct((E, D), jnp.float32),
    mesh=_MESH,
    scratch_types=[
        pltpu.VMEM((CH,), jnp.int32),
        pltpu.VMEM((CH,), jnp.int32),
        pltpu.VMEM((CH, D), jnp.float32),
        pltpu.SemaphoreType.DMA,
    ],
)
def _sc_gather(sd_hbm, ss_hbm, dst_hbm, src_hbm, out_hbm, idx_d, idx_s, rows,
               sem):
    wid = lax.axis_index("s") * NC + lax.axis_index("c")
    base = pl.multiple_of(wid * EPW, 8)

    def body(j, carry):
        off = pl.multiple_of(base + j * CH, 8)
        pltpu.sync_copy(dst_hbm.at[pl.ds(off, CH)], idx_d)
        pltpu.sync_copy(src_hbm.at[pl.ds(off, CH)], idx_s)
        pltpu.async_copy(sd_hbm.at[idx_d], rows, sem).wait()
        pltpu.async_copy(ss_hbm.at[idx_s], rows, sem, add=True).wait()
        pltpu.sync_copy(rows, out_hbm.at[pl.ds(off, CH)])
        return carry

    lax.fori_loop(0, NCHUNK, body, 0)


def _make_sc_scatter(width):
    @functools.partial(
        pl.kernel,
        out_type=jax.ShapeDtypeStruct((NC, NP, width), jnp.float32),
        mesh=_MESH,
        scratch_types=[
            pltpu.VMEM((CH,), jnp.int32),
            pltpu.VMEM((CH, width), jnp.float32),
            pltpu.VMEM_SHARED((NP, width), jnp.float32),
        ],
    )
    def _sc_scatter(m_hbm, dst_hbm, zero_hbm, out_hbm, idx, rows, acc):
        c = lax.axis_index("c")
        s = lax.axis_index("s")
        wid = s * NC + c
        base = pl.multiple_of(wid * EPW, 8)
        pltpu.sync_copy(zero_hbm.at[pl.ds(s * NPS, NPS)],
                        acc.at[pl.ds(s * NPS, NPS)])
        plsc.subcore_barrier()

        def body(j, carry):
            off = pl.multiple_of(base + j * CH, 8)
            pltpu.sync_copy(dst_hbm.at[pl.ds(off, CH)], idx)
            pltpu.sync_copy(m_hbm.at[pl.ds(off, CH)], rows)
            pltpu.sync_copy(rows, acc.at[idx], add=True)
            return carry

        lax.fori_loop(0, NCHUNK, body, 0)
        plsc.subcore_barrier()
        pltpu.sync_copy(acc.at[pl.ds(s * NPS, NPS)],
                        out_hbm.at[c, pl.ds(s * NPS, NPS)])

    return _sc_scatter


_sc_scatter_d = _make_sc_scatter(D)


def _edge_mm_body(g_ref, w2_ref, b2_ref, o_ref):
    g = _silu(g_ref[...])
    z = _mm(g, w2_ref[...]) + b2_ref[...]
    o_ref[...] = _silu(z)


_edge_mm = pl.pallas_call(
    _edge_mm_body,
    grid=(GE,),
    in_specs=[
        pl.BlockSpec((BE, D), lambda i: (i, 0)),
        pl.BlockSpec((D, D), lambda i: (0, 0)),
        pl.BlockSpec((1, D), lambda i: (0, 0)),
    ],
    out_specs=pl.BlockSpec((BE, D), lambda i: (i, 0)),
    out_shape=jax.ShapeDtypeStruct((E, D), jnp.float32),
)


def _enc_body(u_ref, pos_ref, var_ref, w1u, w1p, w1v, b1, w2, b2,
              wxd, wxs, wu, wp, wv, b1m, x_ref, sd_ref, ss_ref):
    u = u_ref[...]
    p = pos_ref[...]
    v = var_ref[...]
    z = _mm(u, w1u[...]) + _mm(p, w1p[...]) + _mm(v, w1v[...]) + b1[...]
    x = _silu(z)
    x = _silu(_mm(x, w2[...]) + b2[...])
    x_ref[...] = x
    t = _mm(u, wu[...]) + _mm(p, wp[...])
    sd_ref[...] = _mm(x, wxd[...]) + t + _mm(v, wv[...]) + b1m[...]
    ss_ref[...] = _mm(x, wxs[...]) - t


_enc = pl.pallas_call(
    _enc_body,
    grid=(GN,),
    in_specs=[
        pl.BlockSpec((BN, TW), lambda i: (i, 0)),
        pl.BlockSpec((BN, 1), lambda i: (i, 0)),
        pl.BlockSpec((BN, NV), lambda i: (i, 0)),
        pl.BlockSpec((TW, D), lambda i: (0, 0)),
        pl.BlockSpec((1, D), lambda i: (0, 0)),
        pl.BlockSpec((NV, D), lambda i: (0, 0)),
        pl.BlockSpec((1, D), lambda i: (0, 0)),
        pl.BlockSpec((D, D), lambda i: (0, 0)),
        pl.BlockSpec((1, D), lambda i: (0, 0)),
        pl.BlockSpec((D, D), lambda i: (0, 0)),
        pl.BlockSpec((D, D), lambda i: (0, 0)),
        pl.BlockSpec((TW, D), lambda i: (0, 0)),
        pl.BlockSpec((1, D), lambda i: (0, 0)),
        pl.BlockSpec((NV, D), lambda i: (0, 0)),
        pl.BlockSpec((1, D), lambda i: (0, 0)),
    ],
    out_specs=[
        pl.BlockSpec((BN, D), lambda i: (i, 0)),
        pl.BlockSpec((BN, D), lambda i: (i, 0)),
        pl.BlockSpec((BN, D), lambda i: (i, 0)),
    ],
    out_shape=[
        jax.ShapeDtypeStruct((N, D), jnp.float32),
        jax.ShapeDtypeStruct((N, D), jnp.float32),
        jax.ShapeDtypeStruct((N, D), jnp.float32),
    ],
)


def _upd_body(x_ref, part_ref, cnt_ref, var_ref, ux, ua, uv, b1, w2, b2,
              h_ref, s1_ref, s2_ref):
    x = x_ref[...]
    p = part_ref[0] + part_ref[1]
    c8 = cnt_ref[...]
    cnt = jnp.maximum(c8[0, :, 0:1] + c8[1, :, 0:1], 1.0)
    agg = p / cnt
    z = _mm(x, ux[...]) + _mm(agg, ua[...]) + _mm(var_ref[...], uv[...]) + b1[...]
    upd = _silu(_mm(_silu(z), w2[...]) + b2[...])
    h = x + upd
    h_ref[...] = h

    @pl.when(pl.program_id(0) == 0)
    def _():
        s1_ref[...] = jnp.zeros_like(s1_ref)
        s2_ref[...] = jnp.zeros_like(s2_ref)

    s1_ref[...] += jnp.sum(h, axis=0, keepdims=True)
    s2_ref[...] += jnp.sum(h * h, axis=0, keepdims=True)


_upd = pl.pallas_call(
    _upd_body,
    grid=(GN,),
    in_specs=[
        pl.BlockSpec((BN, D), lambda i: (i, 0)),
        pl.BlockSpec((NC, BN, D), lambda i: (0, i, 0)),
        pl.BlockSpec((NC, BN, D), lambda i: (0, i, 0)),
        pl.BlockSpec((BN, NV), lambda i: (i, 0)),
        pl.BlockSpec((D, D), lambda i: (0, 0)),
        pl.BlockSpec((D, D), lambda i: (0, 0)),
        pl.BlockSpec((NV, D), lambda i: (0, 0)),
        pl.BlockSpec((1, D), lambda i: (0, 0)),
        pl.BlockSpec((D, D), lambda i: (0, 0)),
        pl.BlockSpec((1, D), lambda i: (0, 0)),
    ],
    out_specs=[
        pl.BlockSpec((BN, D), lambda i: (i, 0)),
        pl.BlockSpec((1, D), lambda i: (0, 0)),
        pl.BlockSpec((1, D), lambda i: (0, 0)),
    ],
    out_shape=[
        jax.ShapeDtypeStruct((N, D), jnp.float32),
        jax.ShapeDtypeStruct((1, D), jnp.float32),
        jax.ShapeDtypeStruct((1, D), jnp.float32),
    ],
)


def _norm_proj_body(h_ref, s1_ref, s2_ref, u_ref, pos_ref, var_ref,
                    wxd, wxs, wu, wp, wv, b1m, x_ref, sd_ref, ss_ref):
    mean = s1_ref[...] / N
    var = s2_ref[...] / N - mean * mean
    inv = lax.rsqrt(var + 1e-5)
    xn = (h_ref[...] - mean) * inv
    x_ref[...] = xn
    t = _mm(u_ref[...], wu[...]) + _mm(pos_ref[...], wp[...])
    sd_ref[...] = _mm(xn, wxd[...]) + t + _mm(var_ref[...], wv[...]) + b1m[...]
    ss_ref[...] = _mm(xn, wxs[...]) - t


_norm_proj = pl.pallas_call(
    _norm_proj_body,
    grid=(GN,),
    in_specs=[
        pl.BlockSpec((BN, D), lambda i: (i, 0)),
        pl.BlockSpec((1, D), lambda i: (0, 0)),
        pl.BlockSpec((1, D), lambda i: (0, 0)),
        pl.BlockSpec((BN, TW), lambda i: (i, 0)),
        pl.BlockSpec((BN, 1), lambda i: (i, 0)),
        pl.BlockSpec((BN, NV), lambda i: (i, 0)),
        pl.BlockSpec((D, D), lambda i: (0, 0)),
        pl.BlockSpec((D, D), lambda i: (0, 0)),
        pl.BlockSpec((TW, D), lambda i: (0, 0)),
        pl.BlockSpec((1, D), lambda i: (0, 0)),
        pl.BlockSpec((NV, D), lambda i: (0, 0)),
        pl.BlockSpec((1, D), lambda i: (0, 0)),
    ],
    out_specs=[
        pl.BlockSpec((BN, D), lambda i: (i, 0)),
        pl.BlockSpec((BN, D), lambda i: (i, 0)),
        pl.BlockSpec((BN, D), lambda i: (i, 0)),
    ],
    out_shape=[
        jax.ShapeDtypeStruct((N, D), jnp.float32),
        jax.ShapeDtypeStruct((N, D), jnp.float32),
        jax.ShapeDtypeStruct((N, D), jnp.float32),
    ],
)


def _norm_dec_body(h_ref, s1_ref, s2_ref, wd, bd, o_ref):
    mean = s1_ref[...] / N
    var = s2_ref[...] / N - mean * mean
    inv = lax.rsqrt(var + 1e-5)
    xn = (h_ref[...] - mean) * inv
    o_ref[...] = _mm(xn, wd[...]) + bd[...]


_norm_dec = pl.pallas_call(
    _norm_dec_body,
    grid=(GN,),
    in_specs=[
        pl.BlockSpec((BN, D), lambda i: (i, 0)),
        pl.BlockSpec((1, D), lambda i: (0, 0)),
        pl.BlockSpec((1, D), lambda i: (0, 0)),
        pl.BlockSpec((D, TW), lambda i: (0, 0)),
        pl.BlockSpec((1, TW), lambda i: (0, 0)),
    ],
    out_specs=pl.BlockSpec((BN, TW), lambda i: (i, 0)),
    out_shape=jax.ShapeDtypeStruct((N, TW), jnp.float32),
)


def kernel(u, pos, variables, enc_W1, enc_b1, enc_W2, enc_b2, msg_W1, msg_b1,
           msg_W2, msg_b2, upd_W1, upd_b1, upd_W2, upd_b2, dec_W, dec_b,
           edge_index):
    src = edge_index[0]
    dst = edge_index[1]

    wxd = msg_W1[:, 0:D, :]
    wxs = msg_W1[:, D:2 * D, :]
    wu = msg_W1[:, 2 * D:2 * D + TW, :]
    wp = msg_W1[:, 2 * D + TW:2 * D + TW + 1, :]
    wv = msg_W1[:, 2 * D + TW + 1:, :]
    uxw = upd_W1[:, 0:D, :]
    uaw = upd_W1[:, D:2 * D, :]
    uvw = upd_W1[:, 2 * D:, :]

    row = lambda b: b.reshape(1, -1)

    zeros_nd = jnp.zeros((NP, D), jnp.float32)
    ones_ed = jnp.ones((E, D), jnp.float32)

    cnt8 = _sc_scatter_d(ones_ed, dst, zeros_nd)[:, :N]

    x, sd, ss = _enc(u, pos, variables,
                     enc_W1[0:TW, :], enc_W1[TW:TW + 1, :], enc_W1[TW + 1:, :],
                     row(enc_b1), enc_W2, row(enc_b2),
                     wxd[0], wxs[0], wu[0], wp[0], wv[0], row(msg_b1[0]))

    for i in range(L):
        g = _sc_gather(sd, ss, dst, src)
        m = _edge_mm(g, msg_W2[i], row(msg_b2[i]))
        part = _sc_scatter_d(m, dst, zeros_nd)[:, :N]
        h, s1, s2 = _upd(x, part, cnt8, variables,
                         uxw[i], uaw[i], uvw[i], row(upd_b1[i]),
                         upd_W2[i], row(upd_b2[i]))
        if i < L - 1:
            x, sd, ss = _norm_proj(h, s1, s2, u, pos, variables,
                                   wxd[i + 1], wxs[i + 1], wu[i + 1],
                                   wp[i + 1], wv[i + 1], row(msg_b1[i + 1]))
        else:
            out = _norm_dec(h, s1, s2, dec_W, row(dec_b))
    return out


# speedup vs baseline: 7.1854x; 1.6672x over previous
import functools

import jax
import jax.numpy as jnp
from jax import lax
from jax.experimental import pallas as pl
from jax.experimental.pallas import tpu as pltpu
from jax.experimental.pallas import tpu_sc as plsc

N = 10000
E = 320000
TW = 25
NV = 1
D = 128
L = 6

NC = 2
NS = 16
NW = NC * NS
EPW = E // NW
SUB = 80
KS = 5
KCH = SUB * KS
NIT = EPW // KCH
SUBS = 40
KSS = 5
KCHS = SUBS * KSS
NITS = EPW // KCHS
NP = 10240
NPS = NP // NS

BN = 1000
GN = N // BN
BE = 2000
GE = E // BE


def _silu(x):
    return x * jax.nn.sigmoid(x)


def _mm(a, b):
    return jax.lax.dot_general(a, b, (((1,), (0,)), ((), ())),
                               preferred_element_type=jnp.float32)


_MESH = plsc.VectorSubcoreMesh(core_axis_name="c", subcore_axis_name="s")


@functools.partial(
    pl.kernel,
    out_type=jax.ShapeDtypeStruct((E, D), jnp.float32),
    mesh=_MESH,
    scratch_types=[
        pltpu.VMEM((KCH,), jnp.int32), pltpu.VMEM((KCH,), jnp.int32),
        pltpu.VMEM((KCH,), jnp.int32), pltpu.VMEM((KCH,), jnp.int32),
        pltpu.VMEM((KCH, D), jnp.float32), pltpu.VMEM((KCH, D), jnp.float32),
        pltpu.SemaphoreType.DMA, pltpu.SemaphoreType.DMA,
        pltpu.SemaphoreType.DMA, pltpu.SemaphoreType.DMA,
        pltpu.SemaphoreType.DMA, pltpu.SemaphoreType.DMA,
    ],
)
def _sc_gather(sd_hbm, ss_hbm, dst_hbm, src_hbm, out_hbm,
               idxd0, idxd1, idxs0, idxs1, buf0, buf1,
               isem0, isem1, gsem0, gsem1, wsem0, wsem1):
    idxd = [idxd0, idxd1]
    idxs = [idxs0, idxs1]
    buf = [buf0, buf1]
    isem = [isem0, isem1]
    gsem = [gsem0, gsem1]
    wsem = [wsem0, wsem1]
    wid = lax.axis_index("s") * NC + lax.axis_index("c")
    base = pl.multiple_of(wid * EPW, 8)

    idesc = [None, None]
    wdesc = [None, None]
    off0 = pl.multiple_of(base, 8)
    idesc[0] = (pltpu.async_copy(dst_hbm.at[pl.ds(off0, KCH)], idxd[0], isem[0]),
                pltpu.async_copy(src_hbm.at[pl.ds(off0, KCH)], idxs[0], isem[0]))
    for j in range(NIT):
        p = j & 1
        q = 1 - p
        if j + 1 < NIT:
            off1 = pl.multiple_of(base + (j + 1) * KCH, 8)
            idesc[q] = (
                pltpu.async_copy(dst_hbm.at[pl.ds(off1, KCH)], idxd[q], isem[q]),
                pltpu.async_copy(src_hbm.at[pl.ds(off1, KCH)], idxs[q], isem[q]))
        if wdesc[p] is not None:
            wdesc[p].wait()
        idesc[p][0].wait()
        idesc[p][1].wait()
        g = [pltpu.async_copy(
                sd_hbm.at[idxd[p].at[pl.ds(r * SUB, SUB)]],
                buf[p].at[pl.ds(r * SUB, SUB)], gsem[p])
             for r in range(KS)]
        for d in g:
            d.wait()
        a = [pltpu.async_copy(
                ss_hbm.at[idxs[p].at[pl.ds(r * SUB, SUB)]],
                buf[p].at[pl.ds(r * SUB, SUB)], gsem[p], add=True)
             for r in range(KS)]
        for d in a:
            d.wait()
        offj = pl.multiple_of(base + j * KCH, 8)
        wdesc[p] = pltpu.async_copy(buf[p], out_hbm.at[pl.ds(offj, KCH)],
                                    wsem[p])
    for d in wdesc:
        if d is not None:
            d.wait()


def _make_sc_scatter(width, use_ones=False):
    scratch = [
        pltpu.VMEM((SUBS,), jnp.int32) for _ in range(2 * KSS)
    ] + [
        pltpu.VMEM((KCHS, width), jnp.float32),
        pltpu.VMEM_SHARED((NP, width), jnp.float32),
        pltpu.SemaphoreType.DMA, pltpu.SemaphoreType.DMA,
        pltpu.SemaphoreType.DMA, pltpu.SemaphoreType.DMA,
    ]

    def body(m_hbm, dst_hbm, zero_hbm, out_hbm, *rest):
        idx = [list(rest[0:KSS]), list(rest[KSS:2 * KSS])]
        buf = rest[2 * KSS]
        acc = rest[2 * KSS + 1]
        isem = [rest[2 * KSS + 2], rest[2 * KSS + 3]]
        msem = rest[2 * KSS + 4]
        ssem = rest[2 * KSS + 5]
        c = lax.axis_index("c")
        s = lax.axis_index("s")
        wid = s * NC + c
        base = pl.multiple_of(wid * EPW, 8)
        pltpu.sync_copy(zero_hbm.at[pl.ds(s * NPS, NPS)],
                        acc.at[pl.ds(s * NPS, NPS)])
        if use_ones:
            pltpu.sync_copy(m_hbm, buf)
        plsc.subcore_barrier()

        def fire_idx(j, q):
            off = pl.multiple_of(base + j * KCHS, 8)
            return [pltpu.async_copy(
                    dst_hbm.at[pl.ds(pl.multiple_of(off + r * SUBS, 8), SUBS)],
                    idx[q][r], isem[q]) for r in range(KSS)]

        def fire_m(j):
            off = pl.multiple_of(base + j * KCHS, 8)
            return pltpu.async_copy(m_hbm.at[pl.ds(off, KCHS)], buf, msem)

        idesc = [None, None]
        sdesc = [None, None]
        idesc[0] = fire_idx(0, 0)
        mdesc = None if use_ones else fire_m(0)
        for j in range(NITS):
            p = j & 1
            q = 1 - p
            if j + 1 < NITS:
                if sdesc[q] is not None:
                    for d in sdesc[q]:
                        d.wait()
                    sdesc[q] = None
                idesc[q] = fire_idx(j + 1, q)
            for d in idesc[p]:
                d.wait()
            if mdesc is not None:
                mdesc.wait()
            sdesc[p] = [pltpu.async_copy(
                            buf.at[pl.ds(r * SUBS, SUBS)],
                            acc.at[idx[p][r]], ssem, add=True)
                        for r in range(KSS)]
            if not use_ones:
                for d in sdesc[p]:
                    d.wait()
                sdesc[p] = None
                if j + 1 < NITS:
                    mdesc = fire_m(j + 1)
        for sd in sdesc:
            if sd is not None:
                for d in sd:
                    d.wait()
        plsc.subcore_barrier()
        pltpu.sync_copy(acc.at[pl.ds(s * NPS, NPS)],
                        out_hbm.at[c, pl.ds(s * NPS, NPS)])

    return functools.partial(
        pl.kernel,
        out_type=jax.ShapeDtypeStruct((NC, NP, width), jnp.float32),
        mesh=_MESH,
        scratch_types=scratch,
    )(body)


_sc_scatter_d = _make_sc_scatter(D)
_sc_counts = _make_sc_scatter(D, use_ones=True)


def _edge_mm_body(g_ref, w2_ref, b2_ref, o_ref):
    g = _silu(g_ref[...])
    z = _mm(g, w2_ref[...]) + b2_ref[...]
    o_ref[...] = _silu(z)


_edge_mm = pl.pallas_call(
    _edge_mm_body,
    grid=(GE,),
    in_specs=[
        pl.BlockSpec((BE, D), lambda i: (i, 0)),
        pl.BlockSpec((D, D), lambda i: (0, 0)),
        pl.BlockSpec((1, D), lambda i: (0, 0)),
    ],
    out_specs=pl.BlockSpec((BE, D), lambda i: (i, 0)),
    out_shape=jax.ShapeDtypeStruct((E, D), jnp.float32),
)


def _enc_body(u_ref, pos_ref, var_ref, w1u, w1p, w1v, b1, w2, b2,
              wxd, wxs, wu, wp, wv, b1m, x_ref, sd_ref, ss_ref):
    u = u_ref[...]
    p = pos_ref[...]
    v = var_ref[...]
    z = _mm(u, w1u[...]) + _mm(p, w1p[...]) + _mm(v, w1v[...]) + b1[...]
    x = _silu(z)
    x = _silu(_mm(x, w2[...]) + b2[...])
    x_ref[...] = x
    t = _mm(u, wu[...]) + _mm(p, wp[...])
    sd_ref[...] = _mm(x, wxd[...]) + t + _mm(v, wv[...]) + b1m[...]
    ss_ref[...] = _mm(x, wxs[...]) - t


_enc = pl.pallas_call(
    _enc_body,
    grid=(GN,),
    in_specs=[
        pl.BlockSpec((BN, TW), lambda i: (i, 0)),
        pl.BlockSpec((BN, 1), lambda i: (i, 0)),
        pl.BlockSpec((BN, NV), lambda i: (i, 0)),
        pl.BlockSpec((TW, D), lambda i: (0, 0)),
        pl.BlockSpec((1, D), lambda i: (0, 0)),
        pl.BlockSpec((NV, D), lambda i: (0, 0)),
        pl.BlockSpec((1, D), lambda i: (0, 0)),
        pl.BlockSpec((D, D), lambda i: (0, 0)),
        pl.BlockSpec((1, D), lambda i: (0, 0)),
        pl.BlockSpec((D, D), lambda i: (0, 0)),
        pl.BlockSpec((D, D), lambda i: (0, 0)),
        pl.BlockSpec((TW, D), lambda i: (0, 0)),
        pl.BlockSpec((1, D), lambda i: (0, 0)),
        pl.BlockSpec((NV, D), lambda i: (0, 0)),
        pl.BlockSpec((1, D), lambda i: (0, 0)),
    ],
    out_specs=[
        pl.BlockSpec((BN, D), lambda i: (i, 0)),
        pl.BlockSpec((BN, D), lambda i: (i, 0)),
        pl.BlockSpec((BN, D), lambda i: (i, 0)),
    ],
    out_shape=[
        jax.ShapeDtypeStruct((N, D), jnp.float32),
        jax.ShapeDtypeStruct((N, D), jnp.float32),
        jax.ShapeDtypeStruct((N, D), jnp.float32),
    ],
)


def _upd_body(x_ref, part_ref, cnt_ref, var_ref, ux, ua, uv, b1, w2, b2,
              h_ref, s1_ref, s2_ref):
    x = x_ref[...]
    p = part_ref[0] + part_ref[1]
    c8 = cnt_ref[...]
    cnt = jnp.maximum(c8[0, :, 0:1] + c8[1, :, 0:1], 1.0)
    agg = p / cnt
    z = _mm(x, ux[...]) + _mm(agg, ua[...]) + _mm(var_ref[...], uv[...]) + b1[...]
    upd = _silu(_mm(_silu(z), w2[...]) + b2[...])
    h = x + upd
    h_ref[...] = h

    @pl.when(pl.program_id(0) == 0)
    def _():
        s1_ref[...] = jnp.zeros_like(s1_ref)
        s2_ref[...] = jnp.zeros_like(s2_ref)

    s1_ref[...] += jnp.sum(h, axis=0, keepdims=True)
    s2_ref[...] += jnp.sum(h * h, axis=0, keepdims=True)


_upd = pl.pallas_call(
    _upd_body,
    grid=(GN,),
    in_specs=[
        pl.BlockSpec((BN, D), lambda i: (i, 0)),
        pl.BlockSpec((NC, BN, D), lambda i: (0, i, 0)),
        pl.BlockSpec((NC, BN, D), lambda i: (0, i, 0)),
        pl.BlockSpec((BN, NV), lambda i: (i, 0)),
        pl.BlockSpec((D, D), lambda i: (0, 0)),
        pl.BlockSpec((D, D), lambda i: (0, 0)),
        pl.BlockSpec((NV, D), lambda i: (0, 0)),
        pl.BlockSpec((1, D), lambda i: (0, 0)),
        pl.BlockSpec((D, D), lambda i: (0, 0)),
        pl.BlockSpec((1, D), lambda i: (0, 0)),
    ],
    out_specs=[
        pl.BlockSpec((BN, D), lambda i: (i, 0)),
        pl.BlockSpec((1, D), lambda i: (0, 0)),
        pl.BlockSpec((1, D), lambda i: (0, 0)),
    ],
    out_shape=[
        jax.ShapeDtypeStruct((N, D), jnp.float32),
        jax.ShapeDtypeStruct((1, D), jnp.float32),
        jax.ShapeDtypeStruct((1, D), jnp.float32),
    ],
)


def _norm_proj_body(h_ref, s1_ref, s2_ref, u_ref, pos_ref, var_ref,
                    wxd, wxs, wu, wp, wv, b1m, x_ref, sd_ref, ss_ref):
    mean = s1_ref[...] / N
    var = s2_ref[...] / N - mean * mean
    inv = lax.rsqrt(var + 1e-5)
    xn = (h_ref[...] - mean) * inv
    x_ref[...] = xn
    t = _mm(u_ref[...], wu[...]) + _mm(pos_ref[...], wp[...])
    sd_ref[...] = _mm(xn, wxd[...]) + t + _mm(var_ref[...], wv[...]) + b1m[...]
    ss_ref[...] = _mm(xn, wxs[...]) - t


_norm_proj = pl.pallas_call(
    _norm_proj_body,
    grid=(GN,),
    in_specs=[
        pl.BlockSpec((BN, D), lambda i: (i, 0)),
        pl.BlockSpec((1, D), lambda i: (0, 0)),
        pl.BlockSpec((1, D), lambda i: (0, 0)),
        pl.BlockSpec((BN, TW), lambda i: (i, 0)),
        pl.BlockSpec((BN, 1), lambda i: (i, 0)),
        pl.BlockSpec((BN, NV), lambda i: (i, 0)),
        pl.BlockSpec((D, D), lambda i: (0, 0)),
        pl.BlockSpec((D, D), lambda i: (0, 0)),
        pl.BlockSpec((TW, D), lambda i: (0, 0)),
        pl.BlockSpec((1, D), lambda i: (0, 0)),
        pl.BlockSpec((NV, D), lambda i: (0, 0)),
        pl.BlockSpec((1, D), lambda i: (0, 0)),
    ],
    out_specs=[
        pl.BlockSpec((BN, D), lambda i: (i, 0)),
        pl.BlockSpec((BN, D), lambda i: (i, 0)),
        pl.BlockSpec((BN, D), lambda i: (i, 0)),
    ],
    out_shape=[
        jax.ShapeDtypeStruct((N, D), jnp.float32),
        jax.ShapeDtypeStruct((N, D), jnp.float32),
        jax.ShapeDtypeStruct((N, D), jnp.float32),
    ],
)


def _norm_dec_body(h_ref, s1_ref, s2_ref, wd, bd, o_ref):
    mean = s1_ref[...] / N
    var = s2_ref[...] / N - mean * mean
    inv = lax.rsqrt(var + 1e-5)
    xn = (h_ref[...] - mean) * inv
    o_ref[...] = _mm(xn, wd[...]) + bd[...]


_norm_dec = pl.pallas_call(
    _norm_dec_body,
    grid=(GN,),
    in_specs=[
        pl.BlockSpec((BN, D), lambda i: (i, 0)),
        pl.BlockSpec((1, D), lambda i: (0, 0)),
        pl.BlockSpec((1, D), lambda i: (0, 0)),
        pl.BlockSpec((D, TW), lambda i: (0, 0)),
        pl.BlockSpec((1, TW), lambda i: (0, 0)),
    ],
    out_specs=pl.BlockSpec((BN, TW), lambda i: (i, 0)),
    out_shape=jax.ShapeDtypeStruct((N, TW), jnp.float32),
)


def kernel(u, pos, variables, enc_W1, enc_b1, enc_W2, enc_b2, msg_W1, msg_b1,
           msg_W2, msg_b2, upd_W1, upd_b1, upd_W2, upd_b2, dec_W, dec_b,
           edge_index):
    src = edge_index[0]
    dst = edge_index[1]

    wxd = msg_W1[:, 0:D, :]
    wxs = msg_W1[:, D:2 * D, :]
    wu = msg_W1[:, 2 * D:2 * D + TW, :]
    wp = msg_W1[:, 2 * D + TW:2 * D + TW + 1, :]
    wv = msg_W1[:, 2 * D + TW + 1:, :]
    uxw = upd_W1[:, 0:D, :]
    uaw = upd_W1[:, D:2 * D, :]
    uvw = upd_W1[:, 2 * D:, :]

    row = lambda b: b.reshape(1, -1)

    zeros_nd = jnp.zeros((NP, D), jnp.float32)
    ones_kch = jnp.ones((KCHS, D), jnp.float32)

    cnt8 = _sc_counts(ones_kch, dst, zeros_nd)[:, :N]

    x, sd, ss = _enc(u, pos, variables,
                     enc_W1[0:TW, :], enc_W1[TW:TW + 1, :], enc_W1[TW + 1:, :],
                     row(enc_b1), enc_W2, row(enc_b2),
                     wxd[0], wxs[0], wu[0], wp[0], wv[0], row(msg_b1[0]))

    for i in range(L):
        g = _sc_gather(sd, ss, dst, src)
        m = _edge_mm(g, msg_W2[i], row(msg_b2[i]))
        part = _sc_scatter_d(m, dst, zeros_nd)[:, :N]
        h, s1, s2 = _upd(x, part, cnt8, variables,
                         uxw[i], uaw[i], uvw[i], row(upd_b1[i]),
                         upd_W2[i], row(upd_b2[i]))
        if i < L - 1:
            x, sd, ss = _norm_proj(h, s1, s2, u, pos, variables,
                                   wxd[i + 1], wxs[i + 1], wu[i + 1],
                                   wp[i + 1], wv[i + 1], row(msg_b1[i + 1]))
        else:
            out = _norm_dec(h, s1, s2, dec_W, row(dec_b))
    return out


# speedup vs baseline: 7.2838x; 1.0137x over previous
import functools

import jax
import jax.numpy as jnp
from jax import lax
from jax.experimental import pallas as pl
from jax.experimental.pallas import tpu as pltpu
from jax.experimental.pallas import tpu_sc as plsc

N = 10000
E = 320000
TW = 25
NV = 1
D = 128
L = 6

NC = 2
NS = 16
NW = NC * NS
EPW = E // NW
SUB = 40
KS = 5
KCH = SUB * KS
NSLOT = 4
NIT = EPW // KCH
SUBS = 40
KSS = 5
KCHS = SUBS * KSS
NITS = EPW // KCHS
NP = 10240
NPS = NP // NS

BN = 1000
GN = N // BN
BE = 2000
GE = E // BE


def _silu(x):
    return x * jax.nn.sigmoid(x)


def _mm(a, b):
    return jax.lax.dot_general(a, b, (((1,), (0,)), ((), ())),
                               preferred_element_type=jnp.float32)


_MESH = plsc.VectorSubcoreMesh(core_axis_name="c", subcore_axis_name="s")


@functools.partial(
    pl.kernel,
    out_type=jax.ShapeDtypeStruct((E, D), jnp.float32),
    mesh=_MESH,
    scratch_types=(
        [pltpu.VMEM((KCH,), jnp.int32) for _ in range(2 * NSLOT)] +
        [pltpu.VMEM((KCH, D), jnp.float32) for _ in range(NSLOT)] +
        [pltpu.SemaphoreType.DMA for _ in range(3 * NSLOT)]
    ),
)
def _sc_gather(sd_hbm, ss_hbm, dst_hbm, src_hbm, out_hbm, *rest):
    idxd = list(rest[0:NSLOT])
    idxs = list(rest[NSLOT:2 * NSLOT])
    buf = list(rest[2 * NSLOT:3 * NSLOT])
    isem = list(rest[3 * NSLOT:4 * NSLOT])
    gsem = list(rest[4 * NSLOT:5 * NSLOT])
    wsem = list(rest[5 * NSLOT:6 * NSLOT])
    wid = lax.axis_index("s") * NC + lax.axis_index("c")
    base = pl.multiple_of(wid * EPW, 8)

    def fire_idx(j):
        p = j % NSLOT
        off = pl.multiple_of(base + j * KCH, 8)
        return (pltpu.async_copy(dst_hbm.at[pl.ds(off, KCH)], idxd[p], isem[p]),
                pltpu.async_copy(src_hbm.at[pl.ds(off, KCH)], idxs[p], isem[p]))

    def fire_sd(j):
        p = j % NSLOT
        return [pltpu.async_copy(
                    sd_hbm.at[idxd[p].at[pl.ds(r * SUB, SUB)]],
                    buf[p].at[pl.ds(r * SUB, SUB)], gsem[p])
                for r in range(KS)]

    def fire_add(j):
        p = j % NSLOT
        return [pltpu.async_copy(
                    ss_hbm.at[idxs[p].at[pl.ds(r * SUB, SUB)]],
                    buf[p].at[pl.ds(r * SUB, SUB)], gsem[p], add=True)
                for r in range(KS)]

    def fire_wb(j):
        p = j % NSLOT
        off = pl.multiple_of(base + j * KCH, 8)
        return pltpu.async_copy(buf[p], out_hbm.at[pl.ds(off, KCH)], wsem[p])

    idesc = {}
    sdesc = {}
    adesc = {}
    wdesc = {}
    for jj in range(min(NSLOT, NIT)):
        idesc[jj] = fire_idx(jj)
    for j in range(NIT + 2):
        if j < NIT:
            if j - NSLOT in wdesc:
                wdesc.pop(j - NSLOT).wait()
            for d in idesc.pop(j):
                d.wait()
            sdesc[j] = fire_sd(j)
        if 0 <= j - 1 < NIT:
            for d in sdesc.pop(j - 1):
                d.wait()
            adesc[j - 1] = fire_add(j - 1)
        if 0 <= j - 2 < NIT:
            for d in adesc.pop(j - 2):
                d.wait()
            wdesc[j - 2] = fire_wb(j - 2)
            if j + 2 < NIT:
                idesc[j + 2] = fire_idx(j + 2)
    for d in wdesc.values():
        d.wait()


def _make_sc_scatter(width, use_ones=False):
    scratch = [
        pltpu.VMEM((SUBS,), jnp.int32) for _ in range(2 * KSS)
    ] + [
        pltpu.VMEM((KCHS, width), jnp.float32),
        pltpu.VMEM_SHARED((NP, width), jnp.float32),
        pltpu.SemaphoreType.DMA, pltpu.SemaphoreType.DMA,
        pltpu.SemaphoreType.DMA, pltpu.SemaphoreType.DMA,
    ]

    def body(m_hbm, dst_hbm, zero_hbm, out_hbm, *rest):
        idx = [list(rest[0:KSS]), list(rest[KSS:2 * KSS])]
        buf = rest[2 * KSS]
        acc = rest[2 * KSS + 1]
        isem = [rest[2 * KSS + 2], rest[2 * KSS + 3]]
        msem = rest[2 * KSS + 4]
        ssem = rest[2 * KSS + 5]
        c = lax.axis_index("c")
        s = lax.axis_index("s")
        wid = s * NC + c
        base = pl.multiple_of(wid * EPW, 8)
        pltpu.sync_copy(zero_hbm.at[pl.ds(s * NPS, NPS)],
                        acc.at[pl.ds(s * NPS, NPS)])
        if use_ones:
            pltpu.sync_copy(m_hbm, buf)
        plsc.subcore_barrier()

        def fire_idx(j, q):
            off = pl.multiple_of(base + j * KCHS, 8)
            return [pltpu.async_copy(
                    dst_hbm.at[pl.ds(pl.multiple_of(off + r * SUBS, 8), SUBS)],
                    idx[q][r], isem[q]) for r in range(KSS)]

        def fire_m(j):
            off = pl.multiple_of(base + j * KCHS, 8)
            return pltpu.async_copy(m_hbm.at[pl.ds(off, KCHS)], buf, msem)

        idesc = [None, None]
        sdesc = [None, None]
        idesc[0] = fire_idx(0, 0)
        mdesc = None if use_ones else fire_m(0)
        for j in range(NITS):
            p = j & 1
            q = 1 - p
            if j + 1 < NITS:
                if sdesc[q] is not None:
                    for d in sdesc[q]:
                        d.wait()
                    sdesc[q] = None
                idesc[q] = fire_idx(j + 1, q)
            for d in idesc[p]:
                d.wait()
            if mdesc is not None:
                mdesc.wait()
            sdesc[p] = [pltpu.async_copy(
                            buf.at[pl.ds(r * SUBS, SUBS)],
                            acc.at[idx[p][r]], ssem, add=True)
                        for r in range(KSS)]
            if not use_ones:
                for d in sdesc[p]:
                    d.wait()
                sdesc[p] = None
                if j + 1 < NITS:
                    mdesc = fire_m(j + 1)
        for sd in sdesc:
            if sd is not None:
                for d in sd:
                    d.wait()
        plsc.subcore_barrier()
        pltpu.sync_copy(acc.at[pl.ds(s * NPS, NPS)],
                        out_hbm.at[c, pl.ds(s * NPS, NPS)])

    return functools.partial(
        pl.kernel,
        out_type=jax.ShapeDtypeStruct((NC, NP, width), jnp.float32),
        mesh=_MESH,
        scratch_types=scratch,
    )(body)


_sc_scatter_d = _make_sc_scatter(D)
_sc_counts = _make_sc_scatter(D, use_ones=True)


def _edge_mm_body(g_ref, w2_ref, b2_ref, o_ref):
    g = _silu(g_ref[...])
    z = _mm(g, w2_ref[...]) + b2_ref[...]
    o_ref[...] = _silu(z)


_edge_mm = pl.pallas_call(
    _edge_mm_body,
    grid=(GE,),
    in_specs=[
        pl.BlockSpec((BE, D), lambda i: (i, 0)),
        pl.BlockSpec((D, D), lambda i: (0, 0)),
        pl.BlockSpec((1, D), lambda i: (0, 0)),
    ],
    out_specs=pl.BlockSpec((BE, D), lambda i: (i, 0)),
    out_shape=jax.ShapeDtypeStruct((E, D), jnp.float32),
)


def _enc_body(u_ref, pos_ref, var_ref, w1u, w1p, w1v, b1, w2, b2,
              wxd, wxs, wu, wp, wv, b1m, x_ref, sd_ref, ss_ref):
    u = u_ref[...]
    p = pos_ref[...]
    v = var_ref[...]
    z = _mm(u, w1u[...]) + _mm(p, w1p[...]) + _mm(v, w1v[...]) + b1[...]
    x = _silu(z)
    x = _silu(_mm(x, w2[...]) + b2[...])
    x_ref[...] = x
    t = _mm(u, wu[...]) + _mm(p, wp[...])
    sd_ref[...] = _mm(x, wxd[...]) + t + _mm(v, wv[...]) + b1m[...]
    ss_ref[...] = _mm(x, wxs[...]) - t


_enc = pl.pallas_call(
    _enc_body,
    grid=(GN,),
    in_specs=[
        pl.BlockSpec((BN, TW), lambda i: (i, 0)),
        pl.BlockSpec((BN, 1), lambda i: (i, 0)),
        pl.BlockSpec((BN, NV), lambda i: (i, 0)),
        pl.BlockSpec((TW, D), lambda i: (0, 0)),
        pl.BlockSpec((1, D), lambda i: (0, 0)),
        pl.BlockSpec((NV, D), lambda i: (0, 0)),
        pl.BlockSpec((1, D), lambda i: (0, 0)),
        pl.BlockSpec((D, D), lambda i: (0, 0)),
        pl.BlockSpec((1, D), lambda i: (0, 0)),
        pl.BlockSpec((D, D), lambda i: (0, 0)),
        pl.BlockSpec((D, D), lambda i: (0, 0)),
        pl.BlockSpec((TW, D), lambda i: (0, 0)),
        pl.BlockSpec((1, D), lambda i: (0, 0)),
        pl.BlockSpec((NV, D), lambda i: (0, 0)),
        pl.BlockSpec((1, D), lambda i: (0, 0)),
    ],
    out_specs=[
        pl.BlockSpec((BN, D), lambda i: (i, 0)),
        pl.BlockSpec((BN, D), lambda i: (i, 0)),
        pl.BlockSpec((BN, D), lambda i: (i, 0)),
    ],
    out_shape=[
        jax.ShapeDtypeStruct((N, D), jnp.float32),
        jax.ShapeDtypeStruct((N, D), jnp.float32),
        jax.ShapeDtypeStruct((N, D), jnp.float32),
    ],
)


def _upd_body(x_ref, part_ref, cnt_ref, var_ref, ux, ua, uv, b1, w2, b2,
              h_ref, s1_ref, s2_ref):
    x = x_ref[...]
    p = part_ref[0] + part_ref[1]
    c8 = cnt_ref[...]
    cnt = jnp.maximum(c8[0, :, 0:1] + c8[1, :, 0:1], 1.0)
    agg = p / cnt
    z = _mm(x, ux[...]) + _mm(agg, ua[...]) + _mm(var_ref[...], uv[...]) + b1[...]
    upd = _silu(_mm(_silu(z), w2[...]) + b2[...])
    h = x + upd
    h_ref[...] = h

    @pl.when(pl.program_id(0) == 0)
    def _():
        s1_ref[...] = jnp.zeros_like(s1_ref)
        s2_ref[...] = jnp.zeros_like(s2_ref)

    s1_ref[...] += jnp.sum(h, axis=0, keepdims=True)
    s2_ref[...] += jnp.sum(h * h, axis=0, keepdims=True)


_upd = pl.pallas_call(
    _upd_body,
    grid=(GN,),
    in_specs=[
        pl.BlockSpec((BN, D), lambda i: (i, 0)),
        pl.BlockSpec((NC, BN, D), lambda i: (0, i, 0)),
        pl.BlockSpec((NC, BN, D), lambda i: (0, i, 0)),
        pl.BlockSpec((BN, NV), lambda i: (i, 0)),
        pl.BlockSpec((D, D), lambda i: (0, 0)),
        pl.BlockSpec((D, D), lambda i: (0, 0)),
        pl.BlockSpec((NV, D), lambda i: (0, 0)),
        pl.BlockSpec((1, D), lambda i: (0, 0)),
        pl.BlockSpec((D, D), lambda i: (0, 0)),
        pl.BlockSpec((1, D), lambda i: (0, 0)),
    ],
    out_specs=[
        pl.BlockSpec((BN, D), lambda i: (i, 0)),
        pl.BlockSpec((1, D), lambda i: (0, 0)),
        pl.BlockSpec((1, D), lambda i: (0, 0)),
    ],
    out_shape=[
        jax.ShapeDtypeStruct((N, D), jnp.float32),
        jax.ShapeDtypeStruct((1, D), jnp.float32),
        jax.ShapeDtypeStruct((1, D), jnp.float32),
    ],
)


def _norm_proj_body(h_ref, s1_ref, s2_ref, u_ref, pos_ref, var_ref,
                    wxd, wxs, wu, wp, wv, b1m, x_ref, sd_ref, ss_ref):
    mean = s1_ref[...] / N
    var = s2_ref[...] / N - mean * mean
    inv = lax.rsqrt(var + 1e-5)
    xn = (h_ref[...] - mean) * inv
    x_ref[...] = xn
    t = _mm(u_ref[...], wu[...]) + _mm(pos_ref[...], wp[...])
    sd_ref[...] = _mm(xn, wxd[...]) + t + _mm(var_ref[...], wv[...]) + b1m[...]
    ss_ref[...] = _mm(xn, wxs[...]) - t


_norm_proj = pl.pallas_call(
    _norm_proj_body,
    grid=(GN,),
    in_specs=[
        pl.BlockSpec((BN, D), lambda i: (i, 0)),
        pl.BlockSpec((1, D), lambda i: (0, 0)),
        pl.BlockSpec((1, D), lambda i: (0, 0)),
        pl.BlockSpec((BN, TW), lambda i: (i, 0)),
        pl.BlockSpec((BN, 1), lambda i: (i, 0)),
        pl.BlockSpec((BN, NV), lambda i: (i, 0)),
        pl.BlockSpec((D, D), lambda i: (0, 0)),
        pl.BlockSpec((D, D), lambda i: (0, 0)),
        pl.BlockSpec((TW, D), lambda i: (0, 0)),
        pl.BlockSpec((1, D), lambda i: (0, 0)),
        pl.BlockSpec((NV, D), lambda i: (0, 0)),
        pl.BlockSpec((1, D), lambda i: (0, 0)),
    ],
    out_specs=[
        pl.BlockSpec((BN, D), lambda i: (i, 0)),
        pl.BlockSpec((BN, D), lambda i: (i, 0)),
        pl.BlockSpec((BN, D), lambda i: (i, 0)),
    ],
    out_shape=[
        jax.ShapeDtypeStruct((N, D), jnp.float32),
        jax.ShapeDtypeStruct((N, D), jnp.float32),
        jax.ShapeDtypeStruct((N, D), jnp.float32),
    ],
)


def _norm_dec_body(h_ref, s1_ref, s2_ref, wd, bd, o_ref):
    mean = s1_ref[...] / N
    var = s2_ref[...] / N - mean * mean
    inv = lax.rsqrt(var + 1e-5)
    xn = (h_ref[...] - mean) * inv
    o_ref[...] = _mm(xn, wd[...]) + bd[...]


_norm_dec = pl.pallas_call(
    _norm_dec_body,
    grid=(GN,),
    in_specs=[
        pl.BlockSpec((BN, D), lambda i: (i, 0)),
        pl.BlockSpec((1, D), lambda i: (0, 0)),
        pl.BlockSpec((1, D), lambda i: (0, 0)),
        pl.BlockSpec((D, TW), lambda i: (0, 0)),
        pl.BlockSpec((1, TW), lambda i: (0, 0)),
    ],
    out_specs=pl.BlockSpec((BN, TW), lambda i: (i, 0)),
    out_shape=jax.ShapeDtypeStruct((N, TW), jnp.float32),
)


def kernel(u, pos, variables, enc_W1, enc_b1, enc_W2, enc_b2, msg_W1, msg_b1,
           msg_W2, msg_b2, upd_W1, upd_b1, upd_W2, upd_b2, dec_W, dec_b,
           edge_index):
    src = edge_index[0]
    dst = edge_index[1]

    wxd = msg_W1[:, 0:D, :]
    wxs = msg_W1[:, D:2 * D, :]
    wu = msg_W1[:, 2 * D:2 * D + TW, :]
    wp = msg_W1[:, 2 * D + TW:2 * D + TW + 1, :]
    wv = msg_W1[:, 2 * D + TW + 1:, :]
    uxw = upd_W1[:, 0:D, :]
    uaw = upd_W1[:, D:2 * D, :]
    uvw = upd_W1[:, 2 * D:, :]

    row = lambda b: b.reshape(1, -1)

    zeros_nd = jnp.zeros((NP, D), jnp.float32)
    ones_kch = jnp.ones((KCHS, D), jnp.float32)

    cnt8 = _sc_counts(ones_kch, dst, zeros_nd)[:, :N]

    x, sd, ss = _enc(u, pos, variables,
                     enc_W1[0:TW, :], enc_W1[TW:TW + 1, :], enc_W1[TW + 1:, :],
                     row(enc_b1), enc_W2, row(enc_b2),
                     wxd[0], wxs[0], wu[0], wp[0], wv[0], row(msg_b1[0]))

    for i in range(L):
        g = _sc_gather(sd, ss, dst, src)
        m = _edge_mm(g, msg_W2[i], row(msg_b2[i]))
        part = _sc_scatter_d(m, dst, zeros_nd)[:, :N]
        h, s1, s2 = _upd(x, part, cnt8, variables,
                         uxw[i], uaw[i], uvw[i], row(upd_b1[i]),
                         upd_W2[i], row(upd_b2[i]))
        if i < L - 1:
            x, sd, ss = _norm_proj(h, s1, s2, u, pos, variables,
                                   wxd[i + 1], wxs[i + 1], wu[i + 1],
                                   wp[i + 1], wv[i + 1], row(msg_b1[i + 1]))
        else:
            out = _norm_dec(h, s1, s2, dec_W, row(dec_b))
    return out


# speedup vs baseline: 7.8260x; 1.0744x over previous
import functools

import jax
import jax.numpy as jnp
from jax import lax
from jax.experimental import pallas as pl
from jax.experimental.pallas import tpu as pltpu
from jax.experimental.pallas import tpu_sc as plsc

N = 10000
E = 320000
TW = 25
NV = 1
D = 128
L = 6

NC = 2
NS = 16
NW = NC * NS
EPW = E // NW
SUB = 40
KS = 5
KCH = SUB * KS
NSLOT = 4
NIT = EPW // KCH
SUBS = 80
KSS = 1
KCHS = SUBS * KSS
NITS = EPW // KCHS
NP = 10240
NPS = NP // NS

BN = 1000
GN = N // BN
BE = 2000
GE = E // BE


def _silu(x):
    return x * jax.nn.sigmoid(x)


def _mm(a, b):
    return jax.lax.dot_general(a, b, (((1,), (0,)), ((), ())),
                               preferred_element_type=jnp.float32)


_MESH = plsc.VectorSubcoreMesh(core_axis_name="c", subcore_axis_name="s")


@functools.partial(
    pl.kernel,
    out_type=jax.ShapeDtypeStruct((E, D), jnp.float32),
    mesh=_MESH,
    scratch_types=(
        [pltpu.VMEM((KCH,), jnp.int32) for _ in range(2 * NSLOT)] +
        [pltpu.VMEM((KCH, D), jnp.float32) for _ in range(NSLOT)] +
        [pltpu.SemaphoreType.DMA for _ in range(3 * NSLOT)]
    ),
)
def _sc_gather(sd_hbm, ss_hbm, dst_hbm, src_hbm, out_hbm, *rest):
    idxd = list(rest[0:NSLOT])
    idxs = list(rest[NSLOT:2 * NSLOT])
    buf = list(rest[2 * NSLOT:3 * NSLOT])
    isem = list(rest[3 * NSLOT:4 * NSLOT])
    gsem = list(rest[4 * NSLOT:5 * NSLOT])
    wsem = list(rest[5 * NSLOT:6 * NSLOT])
    wid = lax.axis_index("s") * NC + lax.axis_index("c")
    base = pl.multiple_of(wid * EPW, 8)

    def fire_idx(j):
        p = j % NSLOT
        off = pl.multiple_of(base + j * KCH, 8)
        return (pltpu.async_copy(dst_hbm.at[pl.ds(off, KCH)], idxd[p], isem[p]),
                pltpu.async_copy(src_hbm.at[pl.ds(off, KCH)], idxs[p], isem[p]))

    def fire_sd(j):
        p = j % NSLOT
        return [pltpu.async_copy(
                    sd_hbm.at[idxd[p].at[pl.ds(r * SUB, SUB)]],
                    buf[p].at[pl.ds(r * SUB, SUB)], gsem[p])
                for r in range(KS)]

    def fire_add(j):
        p = j % NSLOT
        return [pltpu.async_copy(
                    ss_hbm.at[idxs[p].at[pl.ds(r * SUB, SUB)]],
                    buf[p].at[pl.ds(r * SUB, SUB)], gsem[p], add=True)
                for r in range(KS)]

    def fire_wb(j):
        p = j % NSLOT
        off = pl.multiple_of(base + j * KCH, 8)
        return pltpu.async_copy(buf[p], out_hbm.at[pl.ds(off, KCH)], wsem[p])

    idesc = {}
    sdesc = {}
    adesc = {}
    wdesc = {}
    for jj in range(min(NSLOT, NIT)):
        idesc[jj] = fire_idx(jj)
    for j in range(NIT + 2):
        if j < NIT:
            if j - NSLOT in wdesc:
                wdesc.pop(j - NSLOT).wait()
            for d in idesc.pop(j):
                d.wait()
            sdesc[j] = fire_sd(j)
        if 0 <= j - 1 < NIT:
            for d in sdesc.pop(j - 1):
                d.wait()
            adesc[j - 1] = fire_add(j - 1)
        if 0 <= j - 2 < NIT:
            for d in adesc.pop(j - 2):
                d.wait()
            wdesc[j - 2] = fire_wb(j - 2)
            if j + 2 < NIT:
                idesc[j + 2] = fire_idx(j + 2)
    for d in wdesc.values():
        d.wait()


def _make_sc_scatter(dtype, use_ones=False):
    nbuf = 1 if use_ones else NSLOT
    scratch = (
        [pltpu.VMEM((SUBS,), jnp.int32) for _ in range(NSLOT * KSS)] +
        [pltpu.VMEM((SUBS if use_ones else KCHS, D), dtype)
         for _ in range(nbuf)] +
        [pltpu.VMEM_SHARED((NP, D), dtype)] +
        [pltpu.SemaphoreType.DMA for _ in range(3 * NSLOT)]
    )

    def body(m_hbm, dst_hbm, zero_hbm, out_hbm, *rest):
        idx = [list(rest[p * KSS:(p + 1) * KSS]) for p in range(NSLOT)]
        nb = NSLOT * KSS
        if use_ones:
            buf = [rest[nb]] * NSLOT
        else:
            buf = list(rest[nb:nb + nbuf])
        acc = rest[nb + nbuf]
        isem = list(rest[nb + nbuf + 1:nb + nbuf + 1 + NSLOT])
        msem = list(rest[nb + nbuf + 1 + NSLOT:nb + nbuf + 1 + 2 * NSLOT])
        ssem = list(rest[nb + nbuf + 1 + 2 * NSLOT:nb + nbuf + 1 + 3 * NSLOT])
        c = lax.axis_index("c")
        s = lax.axis_index("s")
        wid = s * NC + c
        base = pl.multiple_of(wid * EPW, 8)
        pltpu.sync_copy(zero_hbm.at[pl.ds(s * NPS, NPS)],
                        acc.at[pl.ds(s * NPS, NPS)])
        if use_ones:
            pltpu.sync_copy(m_hbm, buf[0])
        plsc.subcore_barrier()

        def fire_loads(j):
            p = j % NSLOT
            off = pl.multiple_of(base + j * KCHS, 8)
            ds = [pltpu.async_copy(
                    dst_hbm.at[pl.ds(pl.multiple_of(off + r * SUBS, 8), SUBS)],
                    idx[p][r], isem[p]) for r in range(KSS)]
            if not use_ones:
                ds.append(pltpu.async_copy(m_hbm.at[pl.ds(off, KCHS)], buf[p],
                                           msem[p]))
            return ds

        def fire_scatter(j):
            p = j % NSLOT
            if use_ones:
                return [pltpu.async_copy(buf[0], acc.at[idx[p][r]],
                                         ssem[p], add=True)
                        for r in range(KSS)]
            return [pltpu.async_copy(
                        buf[p].at[pl.ds(r * SUBS, SUBS)],
                        acc.at[idx[p][r]], ssem[p], add=True)
                    for r in range(KSS)]

        ldesc = {}
        sdesc = {}
        for jj in range(min(NSLOT, NITS)):
            ldesc[jj] = fire_loads(jj)
        for j in range(NITS + 2):
            if j < NITS:
                for d in ldesc.pop(j):
                    d.wait()
                sdesc[j] = fire_scatter(j)
            if 0 <= j - 2 < NITS:
                for d in sdesc.pop(j - 2):
                    d.wait()
                if j + 2 < NITS:
                    ldesc[j + 2] = fire_loads(j + 2)
        plsc.subcore_barrier()
        pltpu.sync_copy(acc.at[pl.ds(s * NPS, NPS)],
                        out_hbm.at[c, pl.ds(s * NPS, NPS)])

    return functools.partial(
        pl.kernel,
        out_type=jax.ShapeDtypeStruct((NC, NP, D), dtype),
        mesh=_MESH,
        scratch_types=scratch,
    )(body)


_sc_scatter = _make_sc_scatter(jnp.float32)
_sc_counts = _make_sc_scatter(jnp.float32, use_ones=True)


def _edge_mm_body(g_ref, w2_ref, b2_ref, o_ref):
    g = _silu(g_ref[...])
    z = _mm(g, w2_ref[...]) + b2_ref[...]
    o_ref[...] = _silu(z)


_edge_mm = pl.pallas_call(
    _edge_mm_body,
    grid=(GE,),
    in_specs=[
        pl.BlockSpec((BE, D), lambda i: (i, 0)),
        pl.BlockSpec((D, D), lambda i: (0, 0)),
        pl.BlockSpec((1, D), lambda i: (0, 0)),
    ],
    out_specs=pl.BlockSpec((BE, D), lambda i: (i, 0)),
    out_shape=jax.ShapeDtypeStruct((E, D), jnp.float32),
)


def _enc_body(u_ref, pos_ref, var_ref, w1u, w1p, w1v, b1, w2, b2,
              wxd, wxs, wu, wp, wv, b1m, x_ref, sd_ref, ss_ref):
    u = u_ref[...]
    p = pos_ref[...]
    v = var_ref[...]
    z = _mm(u, w1u[...]) + _mm(p, w1p[...]) + _mm(v, w1v[...]) + b1[...]
    x = _silu(z)
    x = _silu(_mm(x, w2[...]) + b2[...])
    x_ref[...] = x
    t = _mm(u, wu[...]) + _mm(p, wp[...])
    sd_ref[...] = _mm(x, wxd[...]) + t + _mm(v, wv[...]) + b1m[...]
    ss_ref[...] = _mm(x, wxs[...]) - t


_enc = pl.pallas_call(
    _enc_body,
    grid=(GN,),
    in_specs=[
        pl.BlockSpec((BN, TW), lambda i: (i, 0)),
        pl.BlockSpec((BN, 1), lambda i: (i, 0)),
        pl.BlockSpec((BN, NV), lambda i: (i, 0)),
        pl.BlockSpec((TW, D), lambda i: (0, 0)),
        pl.BlockSpec((1, D), lambda i: (0, 0)),
        pl.BlockSpec((NV, D), lambda i: (0, 0)),
        pl.BlockSpec((1, D), lambda i: (0, 0)),
        pl.BlockSpec((D, D), lambda i: (0, 0)),
        pl.BlockSpec((1, D), lambda i: (0, 0)),
        pl.BlockSpec((D, D), lambda i: (0, 0)),
        pl.BlockSpec((D, D), lambda i: (0, 0)),
        pl.BlockSpec((TW, D), lambda i: (0, 0)),
        pl.BlockSpec((1, D), lambda i: (0, 0)),
        pl.BlockSpec((NV, D), lambda i: (0, 0)),
        pl.BlockSpec((1, D), lambda i: (0, 0)),
    ],
    out_specs=[
        pl.BlockSpec((BN, D), lambda i: (i, 0)),
        pl.BlockSpec((BN, D), lambda i: (i, 0)),
        pl.BlockSpec((BN, D), lambda i: (i, 0)),
    ],
    out_shape=[
        jax.ShapeDtypeStruct((N, D), jnp.float32),
        jax.ShapeDtypeStruct((N, D), jnp.float32),
        jax.ShapeDtypeStruct((N, D), jnp.float32),
    ],
)


def _upd_body(x_ref, part_ref, cnt_ref, var_ref, ux, ua, uv, b1, w2, b2,
              h_ref, s1_ref, s2_ref):
    x = x_ref[...]
    p = (part_ref[0].astype(jnp.float32) + part_ref[1].astype(jnp.float32))
    c8 = cnt_ref[...]
    cnt = jnp.maximum(c8[0, :, 0:1] + c8[1, :, 0:1], 1.0)
    agg = p / cnt
    z = _mm(x, ux[...]) + _mm(agg, ua[...]) + _mm(var_ref[...], uv[...]) + b1[...]
    upd = _silu(_mm(_silu(z), w2[...]) + b2[...])
    h = x + upd
    h_ref[...] = h

    @pl.when(pl.program_id(0) == 0)
    def _():
        s1_ref[...] = jnp.zeros_like(s1_ref)
        s2_ref[...] = jnp.zeros_like(s2_ref)

    s1_ref[...] += jnp.sum(h, axis=0, keepdims=True)
    s2_ref[...] += jnp.sum(h * h, axis=0, keepdims=True)


_upd = pl.pallas_call(
    _upd_body,
    grid=(GN,),
    in_specs=[
        pl.BlockSpec((BN, D), lambda i: (i, 0)),
        pl.BlockSpec((NC, BN, D), lambda i: (0, i, 0)),
        pl.BlockSpec((NC, BN, D), lambda i: (0, i, 0)),
        pl.BlockSpec((BN, NV), lambda i: (i, 0)),
        pl.BlockSpec((D, D), lambda i: (0, 0)),
        pl.BlockSpec((D, D), lambda i: (0, 0)),
        pl.BlockSpec((NV, D), lambda i: (0, 0)),
        pl.BlockSpec((1, D), lambda i: (0, 0)),
        pl.BlockSpec((D, D), lambda i: (0, 0)),
        pl.BlockSpec((1, D), lambda i: (0, 0)),
    ],
    out_specs=[
        pl.BlockSpec((BN, D), lambda i: (i, 0)),
        pl.BlockSpec((1, D), lambda i: (0, 0)),
        pl.BlockSpec((1, D), lambda i: (0, 0)),
    ],
    out_shape=[
        jax.ShapeDtypeStruct((N, D), jnp.float32),
        jax.ShapeDtypeStruct((1, D), jnp.float32),
        jax.ShapeDtypeStruct((1, D), jnp.float32),
    ],
)


def _norm_proj_body(h_ref, s1_ref, s2_ref, u_ref, pos_ref, var_ref,
                    wxd, wxs, wu, wp, wv, b1m, x_ref, sd_ref, ss_ref):
    mean = s1_ref[...] / N
    var = s2_ref[...] / N - mean * mean
    inv = lax.rsqrt(var + 1e-5)
    xn = (h_ref[...] - mean) * inv
    x_ref[...] = xn
    t = _mm(u_ref[...], wu[...]) + _mm(pos_ref[...], wp[...])
    sd_ref[...] = _mm(xn, wxd[...]) + t + _mm(var_ref[...], wv[...]) + b1m[...]
    ss_ref[...] = _mm(xn, wxs[...]) - t


_norm_proj = pl.pallas_call(
    _norm_proj_body,
    grid=(GN,),
    in_specs=[
        pl.BlockSpec((BN, D), lambda i: (i, 0)),
        pl.BlockSpec((1, D), lambda i: (0, 0)),
        pl.BlockSpec((1, D), lambda i: (0, 0)),
        pl.BlockSpec((BN, TW), lambda i: (i, 0)),
        pl.BlockSpec((BN, 1), lambda i: (i, 0)),
        pl.BlockSpec((BN, NV), lambda i: (i, 0)),
        pl.BlockSpec((D, D), lambda i: (0, 0)),
        pl.BlockSpec((D, D), lambda i: (0, 0)),
        pl.BlockSpec((TW, D), lambda i: (0, 0)),
        pl.BlockSpec((1, D), lambda i: (0, 0)),
        pl.BlockSpec((NV, D), lambda i: (0, 0)),
        pl.BlockSpec((1, D), lambda i: (0, 0)),
    ],
    out_specs=[
        pl.BlockSpec((BN, D), lambda i: (i, 0)),
        pl.BlockSpec((BN, D), lambda i: (i, 0)),
        pl.BlockSpec((BN, D), lambda i: (i, 0)),
    ],
    out_shape=[
        jax.ShapeDtypeStruct((N, D), jnp.float32),
        jax.ShapeDtypeStruct((N, D), jnp.float32),
        jax.ShapeDtypeStruct((N, D), jnp.float32),
    ],
)


def _norm_dec_body(h_ref, s1_ref, s2_ref, wd, bd, o_ref):
    mean = s1_ref[...] / N
    var = s2_ref[...] / N - mean * mean
    inv = lax.rsqrt(var + 1e-5)
    xn = (h_ref[...] - mean) * inv
    o_ref[...] = _mm(xn, wd[...]) + bd[...]


_norm_dec = pl.pallas_call(
    _norm_dec_body,
    grid=(GN,),
    in_specs=[
        pl.BlockSpec((BN, D), lambda i: (i, 0)),
        pl.BlockSpec((1, D), lambda i: (0, 0)),
        pl.BlockSpec((1, D), lambda i: (0, 0)),
        pl.BlockSpec((D, TW), lambda i: (0, 0)),
        pl.BlockSpec((1, TW), lambda i: (0, 0)),
    ],
    out_specs=pl.BlockSpec((BN, TW), lambda i: (i, 0)),
    out_shape=jax.ShapeDtypeStruct((N, TW), jnp.float32),
)


def kernel(u, pos, variables, enc_W1, enc_b1, enc_W2, enc_b2, msg_W1, msg_b1,
           msg_W2, msg_b2, upd_W1, upd_b1, upd_W2, upd_b2, dec_W, dec_b,
           edge_index):
    src = edge_index[0]
    dst = edge_index[1]

    wxd = msg_W1[:, 0:D, :]
    wxs = msg_W1[:, D:2 * D, :]
    wu = msg_W1[:, 2 * D:2 * D + TW, :]
    wp = msg_W1[:, 2 * D + TW:2 * D + TW + 1, :]
    wv = msg_W1[:, 2 * D + TW + 1:, :]
    uxw = upd_W1[:, 0:D, :]
    uaw = upd_W1[:, D:2 * D, :]
    uvw = upd_W1[:, 2 * D:, :]

    row = lambda b: b.reshape(1, -1)

    zeros_f = jnp.zeros((NP, D), jnp.float32)
    ones_kch = jnp.ones((SUBS, D), jnp.float32)

    cnt8 = _sc_counts(ones_kch, dst, zeros_f)[:, :N]

    x, sd, ss = _enc(u, pos, variables,
                     enc_W1[0:TW, :], enc_W1[TW:TW + 1, :], enc_W1[TW + 1:, :],
                     row(enc_b1), enc_W2, row(enc_b2),
                     wxd[0], wxs[0], wu[0], wp[0], wv[0], row(msg_b1[0]))

    for i in range(L):
        g = _sc_gather(sd, ss, dst, src)
        m = _edge_mm(g, msg_W2[i], row(msg_b2[i]))
        part = _sc_scatter(m, dst, zeros_f)[:, :N]
        h, s1, s2 = _upd(x, part, cnt8, variables,
                         uxw[i], uaw[i], uvw[i], row(upd_b1[i]),
                         upd_W2[i], row(upd_b2[i]))
        if i < L - 1:
            x, sd, ss = _norm_proj(h, s1, s2, u, pos, variables,
                                   wxd[i + 1], wxs[i + 1], wu[i + 1],
                                   wp[i + 1], wv[i + 1], row(msg_b1[i + 1]))
        else:
            out = _norm_dec(h, s1, s2, dec_W, row(dec_b))
    return out


# speedup vs baseline: 8.9945x; 1.1493x over previous
import functools

import jax
import jax.numpy as jnp
from jax import lax
from jax.experimental import pallas as pl
from jax.experimental.pallas import tpu as pltpu
from jax.experimental.pallas import tpu_sc as plsc

N = 10000
E = 320000
TW = 25
NV = 1
D = 128
L = 6

NC = 2
NS = 16
NW = NC * NS
EPW = E // NW
SUB = 40
KS = 5
KCH = SUB * KS
NSLOT = 4
NIT = EPW // KCH
SUBS = 80
KSS = 1
KCHS = SUBS * KSS
NITS = EPW // KCHS
NP = 10240
NPS = NP // NS

BN = 1000
GN = N // BN
BE = 2000
GE = E // BE


def _silu(x):
    return x * jax.nn.sigmoid(x)


def _mm(a, b):
    return jax.lax.dot_general(a, b, (((1,), (0,)), ((), ())),
                               preferred_element_type=jnp.float32)


_MESH = plsc.VectorSubcoreMesh(core_axis_name="c", subcore_axis_name="s")


def _make_sc_gather(ne):
  epw = ne // NW
  nit = epw // KCH

  @functools.partial(
      pl.kernel,
      out_type=jax.ShapeDtypeStruct((ne, D), jnp.float32),
      mesh=_MESH,
      scratch_types=(
          [pltpu.VMEM((KCH,), jnp.int32) for _ in range(2 * NSLOT)] +
          [pltpu.VMEM((KCH, D), jnp.float32) for _ in range(NSLOT)] +
          [pltpu.SemaphoreType.DMA for _ in range(3 * NSLOT)]
      ),
  )
  def _sc_gather(sd_hbm, ss_hbm, dst_hbm, src_hbm, out_hbm, *rest):
    NIT = nit
    idxd = list(rest[0:NSLOT])
    idxs = list(rest[NSLOT:2 * NSLOT])
    buf = list(rest[2 * NSLOT:3 * NSLOT])
    isem = list(rest[3 * NSLOT:4 * NSLOT])
    gsem = list(rest[4 * NSLOT:5 * NSLOT])
    wsem = list(rest[5 * NSLOT:6 * NSLOT])
    wid = lax.axis_index("s") * NC + lax.axis_index("c")
    base = pl.multiple_of(wid * epw, 8)

    def fire_idx(j):
        p = j % NSLOT
        off = pl.multiple_of(base + j * KCH, 8)
        return (pltpu.async_copy(dst_hbm.at[pl.ds(off, KCH)], idxd[p], isem[p]),
                pltpu.async_copy(src_hbm.at[pl.ds(off, KCH)], idxs[p], isem[p]))

    def fire_sd(j):
        p = j % NSLOT
        return [pltpu.async_copy(
                    sd_hbm.at[idxd[p].at[pl.ds(r * SUB, SUB)]],
                    buf[p].at[pl.ds(r * SUB, SUB)], gsem[p])
                for r in range(KS)]

    def fire_add(j):
        p = j % NSLOT
        return [pltpu.async_copy(
                    ss_hbm.at[idxs[p].at[pl.ds(r * SUB, SUB)]],
                    buf[p].at[pl.ds(r * SUB, SUB)], gsem[p], add=True)
                for r in range(KS)]

    def fire_wb(j):
        p = j % NSLOT
        off = pl.multiple_of(base + j * KCH, 8)
        return pltpu.async_copy(buf[p], out_hbm.at[pl.ds(off, KCH)], wsem[p])

    idesc = {}
    sdesc = {}
    adesc = {}
    wdesc = {}
    for jj in range(min(NSLOT, NIT)):
        idesc[jj] = fire_idx(jj)
    for j in range(NIT + 2):
        if j < NIT:
            if j - NSLOT in wdesc:
                wdesc.pop(j - NSLOT).wait()
            for d in idesc.pop(j):
                d.wait()
            sdesc[j] = fire_sd(j)
        if 0 <= j - 1 < NIT:
            for d in sdesc.pop(j - 1):
                d.wait()
            adesc[j - 1] = fire_add(j - 1)
        if 0 <= j - 2 < NIT:
            for d in adesc.pop(j - 2):
                d.wait()
            wdesc[j - 2] = fire_wb(j - 2)
            if j + 2 < NIT:
                idesc[j + 2] = fire_idx(j + 2)
    for d in wdesc.values():
        d.wait()

  return _sc_gather


EA = 192000
EB = E - EA
_sc_gather_a = _make_sc_gather(EA)
_sc_gather_b = _make_sc_gather(EB)


def _make_sc_scatter(dtype, ne=E, use_ones=False):
    epw = ne // NW
    nits = epw // KCHS
    nbuf = 1 if use_ones else NSLOT
    scratch = (
        [pltpu.VMEM((SUBS,), jnp.int32) for _ in range(NSLOT * KSS)] +
        [pltpu.VMEM((SUBS if use_ones else KCHS, D), dtype)
         for _ in range(nbuf)] +
        [pltpu.VMEM_SHARED((NP, D), dtype)] +
        [pltpu.SemaphoreType.DMA for _ in range(3 * NSLOT)]
    )

    def body(m_hbm, dst_hbm, zero_hbm, out_hbm, *rest):
        idx = [list(rest[p * KSS:(p + 1) * KSS]) for p in range(NSLOT)]
        nb = NSLOT * KSS
        if use_ones:
            buf = [rest[nb]] * NSLOT
        else:
            buf = list(rest[nb:nb + nbuf])
        acc = rest[nb + nbuf]
        isem = list(rest[nb + nbuf + 1:nb + nbuf + 1 + NSLOT])
        msem = list(rest[nb + nbuf + 1 + NSLOT:nb + nbuf + 1 + 2 * NSLOT])
        ssem = list(rest[nb + nbuf + 1 + 2 * NSLOT:nb + nbuf + 1 + 3 * NSLOT])
        NITS = nits
        c = lax.axis_index("c")
        s = lax.axis_index("s")
        wid = s * NC + c
        base = pl.multiple_of(wid * epw, 8)
        pltpu.sync_copy(zero_hbm.at[pl.ds(s * NPS, NPS)],
                        acc.at[pl.ds(s * NPS, NPS)])
        if use_ones:
            pltpu.sync_copy(m_hbm, buf[0])
        plsc.subcore_barrier()

        def fire_loads(j):
            p = j % NSLOT
            off = pl.multiple_of(base + j * KCHS, 8)
            ds = [pltpu.async_copy(
                    dst_hbm.at[pl.ds(pl.multiple_of(off + r * SUBS, 8), SUBS)],
                    idx[p][r], isem[p]) for r in range(KSS)]
            if not use_ones:
                ds.append(pltpu.async_copy(m_hbm.at[pl.ds(off, KCHS)], buf[p],
                                           msem[p]))
            return ds

        def fire_scatter(j):
            p = j % NSLOT
            if use_ones:
                return [pltpu.async_copy(buf[0], acc.at[idx[p][r]],
                                         ssem[p], add=True)
                        for r in range(KSS)]
            return [pltpu.async_copy(
                        buf[p].at[pl.ds(r * SUBS, SUBS)],
                        acc.at[idx[p][r]], ssem[p], add=True)
                    for r in range(KSS)]

        ldesc = {}
        sdesc = {}
        for jj in range(min(NSLOT, NITS)):
            ldesc[jj] = fire_loads(jj)
        for j in range(NITS + 2):
            if j < NITS:
                for d in ldesc.pop(j):
                    d.wait()
                sdesc[j] = fire_scatter(j)
            if 0 <= j - 2 < NITS:
                for d in sdesc.pop(j - 2):
                    d.wait()
                if j + 2 < NITS:
                    ldesc[j + 2] = fire_loads(j + 2)
        plsc.subcore_barrier()
        pltpu.sync_copy(acc.at[pl.ds(s * NPS, NPS)],
                        out_hbm.at[c, pl.ds(s * NPS, NPS)])

    return functools.partial(
        pl.kernel,
        out_type=jax.ShapeDtypeStruct((NC, NP, D), dtype),
        mesh=_MESH,
        scratch_types=scratch,
    )(body)


_sc_scatter_a = _make_sc_scatter(jnp.float32, EA)
_sc_scatter_b = _make_sc_scatter(jnp.float32, EB)
_sc_counts = _make_sc_scatter(jnp.float32, use_ones=True)


def _edge_mm_body(g_ref, w2_ref, b2_ref, o_ref):
    g = _silu(g_ref[...])
    z = _mm(g, w2_ref[...]) + b2_ref[...]
    o_ref[...] = _silu(z)


def _make_edge_mm(ne):
    return pl.pallas_call(
        _edge_mm_body,
        grid=(ne // BE,),
        in_specs=[
            pl.BlockSpec((BE, D), lambda i: (i, 0)),
            pl.BlockSpec((D, D), lambda i: (0, 0)),
            pl.BlockSpec((1, D), lambda i: (0, 0)),
        ],
        out_specs=pl.BlockSpec((BE, D), lambda i: (i, 0)),
        out_shape=jax.ShapeDtypeStruct((ne, D), jnp.float32),
    )


_edge_mm_a = _make_edge_mm(EA)
_edge_mm_b = _make_edge_mm(EB)


def _enc_body(u_ref, pos_ref, var_ref, w1u, w1p, w1v, b1, w2, b2,
              wxd, wxs, wu, wp, wv, b1m, x_ref, sd_ref, ss_ref):
    u = u_ref[...]
    p = pos_ref[...]
    v = var_ref[...]
    z = _mm(u, w1u[...]) + _mm(p, w1p[...]) + _mm(v, w1v[...]) + b1[...]
    x = _silu(z)
    x = _silu(_mm(x, w2[...]) + b2[...])
    x_ref[...] = x
    t = _mm(u, wu[...]) + _mm(p, wp[...])
    sd_ref[...] = _mm(x, wxd[...]) + t + _mm(v, wv[...]) + b1m[...]
    ss_ref[...] = _mm(x, wxs[...]) - t


_enc = pl.pallas_call(
    _enc_body,
    grid=(GN,),
    in_specs=[
        pl.BlockSpec((BN, TW), lambda i: (i, 0)),
        pl.BlockSpec((BN, 1), lambda i: (i, 0)),
        pl.BlockSpec((BN, NV), lambda i: (i, 0)),
        pl.BlockSpec((TW, D), lambda i: (0, 0)),
        pl.BlockSpec((1, D), lambda i: (0, 0)),
        pl.BlockSpec((NV, D), lambda i: (0, 0)),
        pl.BlockSpec((1, D), lambda i: (0, 0)),
        pl.BlockSpec((D, D), lambda i: (0, 0)),
        pl.BlockSpec((1, D), lambda i: (0, 0)),
        pl.BlockSpec((D, D), lambda i: (0, 0)),
        pl.BlockSpec((D, D), lambda i: (0, 0)),
        pl.BlockSpec((TW, D), lambda i: (0, 0)),
        pl.BlockSpec((1, D), lambda i: (0, 0)),
        pl.BlockSpec((NV, D), lambda i: (0, 0)),
        pl.BlockSpec((1, D), lambda i: (0, 0)),
    ],
    out_specs=[
        pl.BlockSpec((BN, D), lambda i: (i, 0)),
        pl.BlockSpec((BN, D), lambda i: (i, 0)),
        pl.BlockSpec((BN, D), lambda i: (i, 0)),
    ],
    out_shape=[
        jax.ShapeDtypeStruct((N, D), jnp.float32),
        jax.ShapeDtypeStruct((N, D), jnp.float32),
        jax.ShapeDtypeStruct((N, D), jnp.float32),
    ],
)


def _upd_body(x_ref, part_ref, partb_ref, cnt_ref, var_ref, ux, ua, uv, b1,
              w2, b2, h_ref, s1_ref, s2_ref):
    x = x_ref[...]
    p = (part_ref[0] + part_ref[1]) + (partb_ref[0] + partb_ref[1])
    c8 = cnt_ref[...]
    cnt = jnp.maximum(c8[0, :, 0:1] + c8[1, :, 0:1], 1.0)
    agg = p / cnt
    z = _mm(x, ux[...]) + _mm(agg, ua[...]) + _mm(var_ref[...], uv[...]) + b1[...]
    upd = _silu(_mm(_silu(z), w2[...]) + b2[...])
    h = x + upd
    h_ref[...] = h

    @pl.when(pl.program_id(0) == 0)
    def _():
        s1_ref[...] = jnp.zeros_like(s1_ref)
        s2_ref[...] = jnp.zeros_like(s2_ref)

    s1_ref[...] += jnp.sum(h, axis=0, keepdims=True)
    s2_ref[...] += jnp.sum(h * h, axis=0, keepdims=True)


_upd = pl.pallas_call(
    _upd_body,
    grid=(GN,),
    in_specs=[
        pl.BlockSpec((BN, D), lambda i: (i, 0)),
        pl.BlockSpec((NC, BN, D), lambda i: (0, i, 0)),
        pl.BlockSpec((NC, BN, D), lambda i: (0, i, 0)),
        pl.BlockSpec((NC, BN, D), lambda i: (0, i, 0)),
        pl.BlockSpec((BN, NV), lambda i: (i, 0)),
        pl.BlockSpec((D, D), lambda i: (0, 0)),
        pl.BlockSpec((D, D), lambda i: (0, 0)),
        pl.BlockSpec((NV, D), lambda i: (0, 0)),
        pl.BlockSpec((1, D), lambda i: (0, 0)),
        pl.BlockSpec((D, D), lambda i: (0, 0)),
        pl.BlockSpec((1, D), lambda i: (0, 0)),
    ],
    out_specs=[
        pl.BlockSpec((BN, D), lambda i: (i, 0)),
        pl.BlockSpec((1, D), lambda i: (0, 0)),
        pl.BlockSpec((1, D), lambda i: (0, 0)),
    ],
    out_shape=[
        jax.ShapeDtypeStruct((N, D), jnp.float32),
        jax.ShapeDtypeStruct((1, D), jnp.float32),
        jax.ShapeDtypeStruct((1, D), jnp.float32),
    ],
)


def _norm_proj_body(h_ref, s1_ref, s2_ref, u_ref, pos_ref, var_ref,
                    wxd, wxs, wu, wp, wv, b1m, x_ref, sd_ref, ss_ref):
    mean = s1_ref[...] / N
    var = s2_ref[...] / N - mean * mean
    inv = lax.rsqrt(var + 1e-5)
    xn = (h_ref[...] - mean) * inv
    x_ref[...] = xn
    t = _mm(u_ref[...], wu[...]) + _mm(pos_ref[...], wp[...])
    sd_ref[...] = _mm(xn, wxd[...]) + t + _mm(var_ref[...], wv[...]) + b1m[...]
    ss_ref[...] = _mm(xn, wxs[...]) - t


_norm_proj = pl.pallas_call(
    _norm_proj_body,
    grid=(GN,),
    in_specs=[
        pl.BlockSpec((BN, D), lambda i: (i, 0)),
        pl.BlockSpec((1, D), lambda i: (0, 0)),
        pl.BlockSpec((1, D), lambda i: (0, 0)),
        pl.BlockSpec((BN, TW), lambda i: (i, 0)),
        pl.BlockSpec((BN, 1), lambda i: (i, 0)),
        pl.BlockSpec((BN, NV), lambda i: (i, 0)),
        pl.BlockSpec((D, D), lambda i: (0, 0)),
        pl.BlockSpec((D, D), lambda i: (0, 0)),
        pl.BlockSpec((TW, D), lambda i: (0, 0)),
        pl.BlockSpec((1, D), lambda i: (0, 0)),
        pl.BlockSpec((NV, D), lambda i: (0, 0)),
        pl.BlockSpec((1, D), lambda i: (0, 0)),
    ],
    out_specs=[
        pl.BlockSpec((BN, D), lambda i: (i, 0)),
        pl.BlockSpec((BN, D), lambda i: (i, 0)),
        pl.BlockSpec((BN, D), lambda i: (i, 0)),
    ],
    out_shape=[
        jax.ShapeDtypeStruct((N, D), jnp.float32),
        jax.ShapeDtypeStruct((N, D), jnp.float32),
        jax.ShapeDtypeStruct((N, D), jnp.float32),
    ],
)


def _norm_dec_body(h_ref, s1_ref, s2_ref, wd, bd, o_ref):
    mean = s1_ref[...] / N
    var = s2_ref[...] / N - mean * mean
    inv = lax.rsqrt(var + 1e-5)
    xn = (h_ref[...] - mean) * inv
    o_ref[...] = _mm(xn, wd[...]) + bd[...]


_norm_dec = pl.pallas_call(
    _norm_dec_body,
    grid=(GN,),
    in_specs=[
        pl.BlockSpec((BN, D), lambda i: (i, 0)),
        pl.BlockSpec((1, D), lambda i: (0, 0)),
        pl.BlockSpec((1, D), lambda i: (0, 0)),
        pl.BlockSpec((D, TW), lambda i: (0, 0)),
        pl.BlockSpec((1, TW), lambda i: (0, 0)),
    ],
    out_specs=pl.BlockSpec((BN, TW), lambda i: (i, 0)),
    out_shape=jax.ShapeDtypeStruct((N, TW), jnp.float32),
)


def kernel(u, pos, variables, enc_W1, enc_b1, enc_W2, enc_b2, msg_W1, msg_b1,
           msg_W2, msg_b2, upd_W1, upd_b1, upd_W2, upd_b2, dec_W, dec_b,
           edge_index):
    src = edge_index[0]
    dst = edge_index[1]
    dsta, dstb = dst[:EA], dst[EA:]
    srca, srcb = src[:EA], src[EA:]

    wxd = msg_W1[:, 0:D, :]
    wxs = msg_W1[:, D:2 * D, :]
    wu = msg_W1[:, 2 * D:2 * D + TW, :]
    wp = msg_W1[:, 2 * D + TW:2 * D + TW + 1, :]
    wv = msg_W1[:, 2 * D + TW + 1:, :]
    uxw = upd_W1[:, 0:D, :]
    uaw = upd_W1[:, D:2 * D, :]
    uvw = upd_W1[:, 2 * D:, :]

    row = lambda b: b.reshape(1, -1)

    zeros_f = jnp.zeros((NP, D), jnp.float32)
    ones_kch = jnp.ones((SUBS, D), jnp.float32)

    cnt8 = _sc_counts(ones_kch, dst, zeros_f)[:, :N]

    x, sd, ss = _enc(u, pos, variables,
                     enc_W1[0:TW, :], enc_W1[TW:TW + 1, :], enc_W1[TW + 1:, :],
                     row(enc_b1), enc_W2, row(enc_b2),
                     wxd[0], wxs[0], wu[0], wp[0], wv[0], row(msg_b1[0]))

    for i in range(L):
        ga = _sc_gather_a(sd, ss, dsta, srca)
        gb = _sc_gather_b(sd, ss, dstb, srcb)
        ma = _edge_mm_a(ga, msg_W2[i], row(msg_b2[i]))
        mb = _edge_mm_b(gb, msg_W2[i], row(msg_b2[i]))
        parta = _sc_scatter_a(ma, dsta, zeros_f)[:, :N]
        partb = _sc_scatter_b(mb, dstb, zeros_f)[:, :N]
        h, s1, s2 = _upd(x, parta, partb, cnt8, variables,
                         uxw[i], uaw[i], uvw[i], row(upd_b1[i]),
                         upd_W2[i], row(upd_b2[i]))
        if i < L - 1:
            x, sd, ss = _norm_proj(h, s1, s2, u, pos, variables,
                                   wxd[i + 1], wxs[i + 1], wu[i + 1],
                                   wp[i + 1], wv[i + 1], row(msg_b1[i + 1]))
        else:
            out = _norm_dec(h, s1, s2, dec_W, row(dec_b))
    return out
